# baseline (speedup 1.0000x reference)
import jax
import jax.numpy as jnp
from jax.experimental import pallas as pl
from jax.experimental.pallas import tpu as pltpu

_EPS = 1e-5


def kernel(x57, x51, w, gamma, beta):
    N, Cin, H, W = x57.shape
    Cout = w.shape[0]
    HW = H * W
    M_total = N * HW
    inv_m = float(1.0 / M_total)

    x3 = x57.reshape(N, Cin, HW)
    r3 = x51.reshape(N, Cout, HW)
    w_mat = w.reshape(Cout, Cin)
    g2 = gamma.reshape(Cout, 1).astype(jnp.float32)
    b2 = beta.reshape(Cout, 1).astype(jnp.float32)

    group = next(gg for gg in (4, 2, 1) if N % gg == 0)
    G = N // group

    def body(x_ref, w_ref, g_ref, b_ref, r_ref, o_ref, y_scr, s_scr, q_scr):
        p = pl.program_id(0)
        j = pl.program_id(1)

        @pl.when(p == 0)
        def _compute():
            @pl.when(j == 0)
            def _init():
                s_scr[...] = jnp.zeros_like(s_scr)
                q_scr[...] = jnp.zeros_like(q_scr)

            wb = w_ref[...].astype(jnp.bfloat16)
            ps = jnp.zeros((Cout, 1), jnp.float32)
            pq = jnp.zeros((Cout, 1), jnp.float32)
            for i in range(group):
                y = jnp.dot(wb, x_ref[i].astype(jnp.bfloat16),
                            preferred_element_type=jnp.float32)
                y_scr[j * group + i] = y
                ps = ps + jnp.sum(y, axis=1, keepdims=True)
                pq = pq + jnp.sum(y * y, axis=1, keepdims=True)
            s_scr[...] += ps
            q_scr[...] += pq

        @pl.when(p == 1)
        def _normalize():
            mean = s_scr[...] * inv_m
            var = jnp.maximum(q_scr[...] * inv_m - mean * mean, 0.0)
            scale = g_ref[...] * jax.lax.rsqrt(var + jnp.float32(_EPS))
            shift = b_ref[...] - mean * scale
            for i in range(group):
                o_ref[i] = y_scr[j * group + i] * scale + shift + r_ref[i]

    out3 = pl.pallas_call(
        body,
        out_shape=jax.ShapeDtypeStruct((N, Cout, HW), jnp.float32),
        grid=(2, G),
        in_specs=[
            pl.BlockSpec((group, Cin, HW),
                         lambda p, j: (jnp.where(p == 0, j, G - 1), 0, 0)),
            pl.BlockSpec((Cout, Cin), lambda p, j: (0, 0)),
            pl.BlockSpec((Cout, 1), lambda p, j: (0, 0)),
            pl.BlockSpec((Cout, 1), lambda p, j: (0, 0)),
            pl.BlockSpec((group, Cout, HW),
                         lambda p, j: (jnp.where(p == 1, j, 0), 0, 0)),
        ],
        out_specs=pl.BlockSpec((group, Cout, HW),
                               lambda p, j: (jnp.where(p == 1, j, 0), 0, 0)),
        scratch_shapes=[
            pltpu.VMEM((N, Cout, HW), jnp.float32),
            pltpu.VMEM((Cout, 1), jnp.float32),
            pltpu.VMEM((Cout, 1), jnp.float32),
        ],
        compiler_params=pltpu.CompilerParams(
            dimension_semantics=("arbitrary", "arbitrary")),
        cost_estimate=pl.CostEstimate(
            flops=2 * M_total * Cin * Cout + 7 * M_total * Cout,
            transcendentals=Cout,
            bytes_accessed=4 * M_total * Cin + 8 * M_total * Cout
            + 4 * Cin * Cout + 16 * Cout),
    )(x3, w_mat, g2, b2, r3)

    return out3.reshape(N, Cout, H, W)


# speedup vs baseline: 1.0089x; 1.0089x over previous
import jax
import jax.numpy as jnp
from jax.experimental import pallas as pl
from jax.experimental.pallas import tpu as pltpu

_EPS = 1e-5


def kernel(x57, x51, w, gamma, beta):
    N, Cin, H, W = x57.shape
    Cout = w.shape[0]
    HW = H * W
    M_total = N * HW
    inv_m = float(1.0 / M_total)

    x3 = x57.reshape(N, Cin, HW)
    r3 = x51.reshape(N, Cout, HW)
    w_mat = w.reshape(Cout, Cin)
    g2 = gamma.reshape(Cout, 1).astype(jnp.float32)
    b2 = beta.reshape(Cout, 1).astype(jnp.float32)

    group = next(gg for gg in (8, 4, 2, 1) if N % gg == 0)
    G = N // group

    def body(x_ref, w_ref, g_ref, b_ref, r_ref, o_ref, y_scr, s_scr, q_scr):
        p = pl.program_id(0)
        j = pl.program_id(1)

        @pl.when(p == 0)
        def _compute():
            @pl.when(j == 0)
            def _init():
                s_scr[...] = jnp.zeros_like(s_scr)
                q_scr[...] = jnp.zeros_like(q_scr)

            wb = w_ref[...].astype(jnp.bfloat16)
            ps = jnp.zeros((Cout, 1), jnp.float32)
            pq = jnp.zeros((Cout, 1), jnp.float32)
            for i in range(group):
                y = jnp.dot(wb, x_ref[i].astype(jnp.bfloat16),
                            preferred_element_type=jnp.float32)
                y_scr[j * group + i] = y
                ps = ps + jnp.sum(y, axis=1, keepdims=True)
                pq = pq + jnp.sum(y * y, axis=1, keepdims=True)
            s_scr[...] += ps
            q_scr[...] += pq

        @pl.when(p == 1)
        def _normalize():
            mean = s_scr[...] * inv_m
            var = jnp.maximum(q_scr[...] * inv_m - mean * mean, 0.0)
            scale = g_ref[...] * jax.lax.rsqrt(var + jnp.float32(_EPS))
            shift = b_ref[...] - mean * scale
            for i in range(group):
                o_ref[i] = y_scr[j * group + i] * scale + shift + r_ref[i]

    out3 = pl.pallas_call(
        body,
        out_shape=jax.ShapeDtypeStruct((N, Cout, HW), jnp.float32),
        grid=(2, G),
        in_specs=[
            pl.BlockSpec((group, Cin, HW),
                         lambda p, j: (jnp.where(p == 0, j, G - 1), 0, 0)),
            pl.BlockSpec((Cout, Cin), lambda p, j: (0, 0)),
            pl.BlockSpec((Cout, 1), lambda p, j: (0, 0)),
            pl.BlockSpec((Cout, 1), lambda p, j: (0, 0)),
            pl.BlockSpec((group, Cout, HW),
                         lambda p, j: (jnp.where(p == 1, j, 0), 0, 0)),
        ],
        out_specs=pl.BlockSpec((group, Cout, HW),
                               lambda p, j: (jnp.where(p == 1, j, 0), 0, 0)),
        scratch_shapes=[
            pltpu.VMEM((N, Cout, HW), jnp.float32),
            pltpu.VMEM((Cout, 1), jnp.float32),
            pltpu.VMEM((Cout, 1), jnp.float32),
        ],
        compiler_params=pltpu.CompilerParams(
            dimension_semantics=("arbitrary", "arbitrary")),
        cost_estimate=pl.CostEstimate(
            flops=2 * M_total * Cin * Cout + 7 * M_total * Cout,
            transcendentals=Cout,
            bytes_accessed=4 * M_total * Cin + 8 * M_total * Cout
            + 4 * Cin * Cout + 16 * Cout),
    )(x3, w_mat, g2, b2, r3)

    return out3.reshape(N, Cout, H, W)


# speedup vs baseline: 1.0191x; 1.0101x over previous
import jax
import jax.numpy as jnp
from jax.experimental import pallas as pl
from jax.experimental.pallas import tpu as pltpu

_EPS = 1e-5


def kernel(x57, x51, w, gamma, beta):
    N, Cin, H, W = x57.shape
    Cout = w.shape[0]
    HW = H * W
    M_total = N * HW
    inv_m = float(1.0 / M_total)

    x3 = x57.reshape(N, Cin, HW)
    r3 = x51.reshape(N, Cout, HW)
    w_mat = w.reshape(Cout, Cin)
    g2 = gamma.reshape(Cout, 1).astype(jnp.float32)
    b2 = beta.reshape(Cout, 1).astype(jnp.float32)

    group = next(gg for gg in (4, 2, 1) if N % gg == 0)
    G = N // group

    def body(x_ref, w_ref, g_ref, b_ref, r_hbm, o_ref,
             y_scr, r_scr, s_scr, q_scr, r_sem):
        p = pl.program_id(0)
        j = pl.program_id(1)
        r_copy = pltpu.make_async_copy(r_hbm, r_scr, r_sem)

        @pl.when(p == 0)
        def _compute():
            @pl.when(j == 0)
            def _init():
                s_scr[...] = jnp.zeros_like(s_scr)
                q_scr[...] = jnp.zeros_like(q_scr)
                r_copy.start()

            wb = w_ref[...].astype(jnp.bfloat16)
            ps = jnp.zeros((Cout, 1), jnp.float32)
            pq = jnp.zeros((Cout, 1), jnp.float32)
            for i in range(group):
                y = jnp.dot(wb, x_ref[i].astype(jnp.bfloat16),
                            preferred_element_type=jnp.float32)
                y_scr[j * group + i] = y
                ps = ps + jnp.sum(y, axis=1, keepdims=True)
                pq = pq + jnp.sum(y * y, axis=1, keepdims=True)
            s_scr[...] += ps
            q_scr[...] += pq

        @pl.when(p == 1)
        def _normalize():
            @pl.when(j == 0)
            def _wait():
                r_copy.wait()

            mean = s_scr[...] * inv_m
            var = jnp.maximum(q_scr[...] * inv_m - mean * mean, 0.0)
            scale = g_ref[...] * jax.lax.rsqrt(var + jnp.float32(_EPS))
            shift = b_ref[...] - mean * scale
            for i in range(group):
                o_ref[i] = (y_scr[j * group + i] * scale + shift
                            + r_scr[j * group + i])

    out3 = pl.pallas_call(
        body,
        out_shape=jax.ShapeDtypeStruct((N, Cout, HW), jnp.float32),
        grid=(2, G),
        in_specs=[
            pl.BlockSpec((group, Cin, HW),
                         lambda p, j: (jnp.where(p == 0, j, G - 1), 0, 0)),
            pl.BlockSpec((Cout, Cin), lambda p, j: (0, 0)),
            pl.BlockSpec((Cout, 1), lambda p, j: (0, 0)),
            pl.BlockSpec((Cout, 1), lambda p, j: (0, 0)),
            pl.BlockSpec(memory_space=pl.ANY),
        ],
        out_specs=pl.BlockSpec((group, Cout, HW),
                               lambda p, j: (jnp.where(p == 1, j, 0), 0, 0)),
        scratch_shapes=[
            pltpu.VMEM((N, Cout, HW), jnp.float32),
            pltpu.VMEM((N, Cout, HW), jnp.float32),
            pltpu.VMEM((Cout, 1), jnp.float32),
            pltpu.VMEM((Cout, 1), jnp.float32),
            pltpu.SemaphoreType.DMA,
        ],
        compiler_params=pltpu.CompilerParams(
            dimension_semantics=("arbitrary", "arbitrary")),
        cost_estimate=pl.CostEstimate(
            flops=2 * M_total * Cin * Cout + 7 * M_total * Cout,
            transcendentals=Cout,
            bytes_accessed=4 * M_total * Cin + 8 * M_total * Cout
            + 4 * Cin * Cout + 16 * Cout),
    )(x3, w_mat, g2, b2, r3)

    return out3.reshape(N, Cout, H, W)
